# Initial kernel scaffold; baseline (speedup 1.0000x reference)
import jax
import jax.numpy as jnp
from jax.experimental import pallas as pl


def kernel(x, edge_index, batch, W1, b1, gamma, beta, W2, b2, W3, b3, W4, b4, Wm1, bm1, Wm2, bm2, Wm3, bm3):
    raise NotImplementedError("write your pallas kernel here")


# speedup vs baseline: 8.2165x; 8.2165x over previous
import functools

import jax
import jax.numpy as jnp
from jax import lax
from jax.experimental import pallas as pl
from jax.experimental.pallas import tpu as pltpu
from jax.experimental.pallas import tpu_sc as plsc

N = 10000
E = 320000
F_IN = 128
H = 64
B = 64
C = 10

NC = 2
NS = 16
NW = NC * NS
EPW = E // NW
CHUNK = 80
NCH = EPW // CHUNK
N_PAD = 10240
RPT = N_PAD // NS

ROW_BLK = 2000
NBLK = N // ROW_BLK


def _elu(v):
    return jnp.where(v > 0, v, jnp.exp(v) - 1.0)


_sc_mesh = plsc.VectorSubcoreMesh(core_axis_name="c", subcore_axis_name="s")


@functools.partial(
    pl.kernel,
    out_type=jax.ShapeDtypeStruct((NC, N_PAD, H), jnp.float32),
    mesh=_sc_mesh,
    scratch_types=[
        pltpu.VMEM((NCH, CHUNK), jnp.int32),
        pltpu.VMEM((NCH, CHUNK), jnp.int32),
        pltpu.VMEM((CHUNK, H), jnp.float32),
        pltpu.SemaphoreType.DMA,
        pltpu.VMEM_SHARED((N_PAD, H), jnp.float32),
    ],
    compiler_params=pltpu.CompilerParams(use_tc_tiling_on_sc=False),
)
def _seg_sum(table, src3, dst3, zeros, out, src_v, dst_v, rows_v, gsem, acc):
    c = lax.axis_index("c")
    s = lax.axis_index("s")
    wid = s * NC + c
    pltpu.sync_copy(src3.at[wid], src_v)
    pltpu.sync_copy(dst3.at[wid], dst_v)
    pltpu.sync_copy(zeros.at[pl.ds(s * RPT, RPT)], acc.at[pl.ds(s * RPT, RPT)])
    plsc.subcore_barrier()

    def chunk(j, carry):
        pltpu.async_copy(table.at[src_v.at[j]], rows_v, gsem).wait()
        pltpu.sync_copy(rows_v, acc.at[dst_v.at[j]], add=True)
        return carry

    lax.fori_loop(0, NCH, chunk, 0)
    plsc.subcore_barrier()
    pltpu.sync_copy(acc.at[pl.ds(s * RPT, RPT)], out.at[c, pl.ds(s * RPT, RPT)])


def _mm_body(x_ref, w_ref, o_ref):
    o_ref[...] = jnp.dot(x_ref[...], w_ref[...],
                         preferred_element_type=jnp.float32)


def _conv1_body(y_ref, s_ref, gs_ref, beta_ref, b1_ref, w2_ref, b2_ref, o_ref):
    h1 = y_ref[...] + s_ref[0] + s_ref[1] + b1_ref[...]
    hb = gs_ref[...] * h1 + beta_ref[...]
    ha = _elu(hb)
    o_ref[...] = _elu(jnp.dot(ha, w2_ref[...],
                              preferred_element_type=jnp.float32) + b2_ref[...])


def _conv2_pool_body(h_ref, s_ref, batch_ref, w3_ref, b3_ref, w4_ref, b4_ref,
                     wm1_ref, bm1_ref, wm2_ref, bm2_ref, wm3_ref, bm3_ref,
                     o_ref, g_acc):
    i = pl.program_id(0)
    h2 = h_ref[...] + s_ref[0] + s_ref[1]
    h2 = _elu(jnp.dot(h2, w3_ref[...],
                      preferred_element_type=jnp.float32) + b3_ref[...])
    h2 = _elu(jnp.dot(h2, w4_ref[...],
                      preferred_element_type=jnp.float32) + b4_ref[...])
    bb = batch_ref[0, 0, :]
    mask_t = (lax.broadcasted_iota(jnp.int32, (B, ROW_BLK), 0)
              == bb[None, :]).astype(jnp.float32)
    part = jnp.dot(mask_t, h2, preferred_element_type=jnp.float32)

    @pl.when(i == 0)
    def _():
        g_acc[...] = part

    @pl.when(i > 0)
    def _():
        g_acc[...] += part

    @pl.when(i == pl.num_programs(0) - 1)
    def _():
        g = g_acc[...]
        o1 = _elu(jnp.dot(g, wm1_ref[...],
                          preferred_element_type=jnp.float32) + bm1_ref[...])
        o2 = _elu(jnp.dot(o1, wm2_ref[...],
                          preferred_element_type=jnp.float32) + bm2_ref[...])
        o = jnp.dot(o2, wm3_ref[...],
                    preferred_element_type=jnp.float32) + bm3_ref[...]
        m = jnp.max(o, axis=-1, keepdims=True)
        z = o - m
        o_ref[...] = z - jnp.log(jnp.sum(jnp.exp(z), axis=-1, keepdims=True))


def _full(shape):
    return pl.BlockSpec(shape, lambda i: tuple(0 for _ in shape))


def kernel(x, edge_index, batch, W1, b1, gamma, beta, W2, b2, W3, b3, W4, b4,
           Wm1, bm1, Wm2, bm2, Wm3, bm3):
    src3 = edge_index[0].astype(jnp.int32).reshape(NW, NCH, CHUNK)
    dst3 = edge_index[1].astype(jnp.int32).reshape(NW, NCH, CHUNK)
    batch3 = batch.astype(jnp.int32).reshape(NBLK, 1, ROW_BLK)
    zeros = jnp.zeros((N_PAD, H), jnp.float32)
    gs = (gamma / jnp.sqrt(1.0 + 1e-5)).reshape(1, H)
    beta2 = beta.reshape(1, H)
    b1r, b2r, b3r, b4r = (v.reshape(1, H) for v in (b1, b2, b3, b4))
    bm1r = bm1.reshape(1, H)
    bm2r = bm2.reshape(1, H // 2)
    bm3r = bm3.reshape(1, C)

    y = pl.pallas_call(
        _mm_body,
        grid=(NBLK,),
        in_specs=[pl.BlockSpec((ROW_BLK, F_IN), lambda i: (i, 0)),
                  _full((F_IN, H))],
        out_specs=pl.BlockSpec((ROW_BLK, H), lambda i: (i, 0)),
        out_shape=jax.ShapeDtypeStruct((N, H), jnp.float32),
    )(x, W1)

    s1 = _seg_sum(y, src3, dst3, zeros)

    h = pl.pallas_call(
        _conv1_body,
        grid=(NBLK,),
        in_specs=[pl.BlockSpec((ROW_BLK, H), lambda i: (i, 0)),
                  pl.BlockSpec((NC, ROW_BLK, H), lambda i: (0, i, 0)),
                  _full((1, H)), _full((1, H)), _full((1, H)),
                  _full((H, H)), _full((1, H))],
        out_specs=pl.BlockSpec((ROW_BLK, H), lambda i: (i, 0)),
        out_shape=jax.ShapeDtypeStruct((N, H), jnp.float32),
    )(y, s1, gs, beta2, b1r, W2, b2r)

    s2 = _seg_sum(h, src3, dst3, zeros)

    out = pl.pallas_call(
        _conv2_pool_body,
        grid=(NBLK,),
        in_specs=[pl.BlockSpec((ROW_BLK, H), lambda i: (i, 0)),
                  pl.BlockSpec((NC, ROW_BLK, H), lambda i: (0, i, 0)),
                  pl.BlockSpec((1, 1, ROW_BLK), lambda i: (i, 0, 0)),
                  _full((H, H)), _full((1, H)),
                  _full((H, H)), _full((1, H)),
                  _full((H, H)), _full((1, H)),
                  _full((H, H // 2)), _full((1, H // 2)),
                  _full((H // 2, C)), _full((1, C))],
        out_specs=_full((B, C)),
        out_shape=jax.ShapeDtypeStruct((B, C), jnp.float32),
        scratch_shapes=[pltpu.VMEM((B, H), jnp.float32)],
    )(h, s2, batch3, W3, b3r, W4, b4r, Wm1, bm1r, Wm2, bm2r, Wm3, bm3r)

    return out
